# Initial kernel scaffold; baseline (speedup 1.0000x reference)
import jax
import jax.numpy as jnp
from jax.experimental import pallas as pl


def kernel(x, edge_index, edge_type, basis1, comb1, q1, k1, b1, basis2, comb2, q2, k2, b2, basis3, comb3, q3, k3, b3):
    raise NotImplementedError("write your pallas kernel here")


# speedup vs baseline: 12.4204x; 12.4204x over previous
import functools

import jax
import jax.numpy as jnp
from jax import lax
from jax.experimental import pallas as pl
from jax.experimental.pallas import tpu as pltpu
from jax.experimental.pallas import tpu_sc as plsc

N = 10000
E = 320000
R = 8
LANES = 16

NC = 2
NS = 16
EPT = E // NS
CH = 80
NCHUNK = EPT // CH

BLK = 400
NBLK = N // BLK

NPAD = 10240
RPS = NPAD // NS
ZR = RPS // 5


def _tc_first_body(x_ref, wlo_ref, whi_ref, wq_ref, wk_ref,
                   xw_ref, sq_ref, sk_ref):
    h = x_ref[...]
    xw_ref[0] = jnp.dot(h, wlo_ref[...], preferred_element_type=jnp.float32)
    xw_ref[1] = jnp.dot(h, whi_ref[...], preferred_element_type=jnp.float32)
    sq_ref[...] = jnp.dot(h, wq_ref[...], preferred_element_type=jnp.float32)
    sk_ref[...] = jnp.dot(h, wk_ref[...], preferred_element_type=jnp.float32)


def _epilogue(num_ref, den_ref, b_ref, c_head, relu):
    nb = jnp.concatenate([num_ref[0], num_ref[1]], axis=-1)
    f_in = nb.shape[-1]
    db = den_ref[...]
    sel = (lax.broadcasted_iota(jnp.int32, (LANES, f_in), 0)
           == lax.broadcasted_iota(jnp.int32, (LANES, f_in), 1) // c_head
           ).astype(jnp.float32)
    de = jnp.dot(db, sel, preferred_element_type=jnp.float32)
    h = nb / (de + 1e-16) + b_ref[...]
    if relu:
        h = jnp.maximum(h, 0.0)
    return h


def _tc_mid_body(c_head, num_ref, den_ref, b_ref, wlo_ref, whi_ref,
                 wq_ref, wk_ref, xw_ref, sq_ref, sk_ref):
    h = _epilogue(num_ref, den_ref, b_ref, c_head, True)
    xw_ref[0] = jnp.dot(h, wlo_ref[...], preferred_element_type=jnp.float32)
    xw_ref[1] = jnp.dot(h, whi_ref[...], preferred_element_type=jnp.float32)
    sq_ref[...] = jnp.dot(h, wq_ref[...], preferred_element_type=jnp.float32)
    sk_ref[...] = jnp.dot(h, wk_ref[...], preferred_element_type=jnp.float32)


def _tc_final_body(c_head, num_ref, den_ref, b_ref, out_ref):
    out_ref[...] = _epilogue(num_ref, den_ref, b_ref, c_head, False)


def _xw_sq_sk_specs(f_out):
    half = f_out // 2
    out_specs = [
        pl.BlockSpec((NC, BLK, R * half), lambda i: (0, i, 0)),
        pl.BlockSpec((BLK, R * LANES), lambda i: (i, 0)),
        pl.BlockSpec((BLK, R * LANES), lambda i: (i, 0)),
    ]
    out_shape = [
        jax.ShapeDtypeStruct((NC, N, R * half), jnp.float32),
        jax.ShapeDtypeStruct((N, R * LANES), jnp.float32),
        jax.ShapeDtypeStruct((N, R * LANES), jnp.float32),
    ]
    return out_specs, out_shape


def _tc_first(x, wlo, whi, wq, wk, f_out):
    out_specs, out_shape = _xw_sq_sk_specs(f_out)
    return pl.pallas_call(
        _tc_first_body,
        grid=(NBLK,),
        in_specs=[
            pl.BlockSpec((BLK, x.shape[1]), lambda i: (i, 0)),
            pl.BlockSpec(wlo.shape, lambda i: (0, 0)),
            pl.BlockSpec(whi.shape, lambda i: (0, 0)),
            pl.BlockSpec(wq.shape, lambda i: (0, 0)),
            pl.BlockSpec(wk.shape, lambda i: (0, 0)),
        ],
        out_specs=out_specs,
        out_shape=out_shape,
    )(x, wlo, whi, wq, wk)


def _tc_mid(num, den, b, wlo, whi, wq, wk, c_head, f_out):
    half_in = num.shape[-1]
    out_specs, out_shape = _xw_sq_sk_specs(f_out)
    return pl.pallas_call(
        functools.partial(_tc_mid_body, c_head),
        grid=(NBLK,),
        in_specs=[
            pl.BlockSpec((NC, BLK, half_in), lambda i: (0, i, 0)),
            pl.BlockSpec((BLK, LANES), lambda i: (i, 0)),
            pl.BlockSpec((1, NC * half_in), lambda i: (0, 0)),
            pl.BlockSpec(wlo.shape, lambda i: (0, 0)),
            pl.BlockSpec(whi.shape, lambda i: (0, 0)),
            pl.BlockSpec(wq.shape, lambda i: (0, 0)),
            pl.BlockSpec(wk.shape, lambda i: (0, 0)),
        ],
        out_specs=out_specs,
        out_shape=out_shape,
    )(num, den, b, wlo, whi, wq, wk)


def _tc_final(num, den, b, c_head):
    half_in = num.shape[-1]
    f_in = NC * half_in
    return pl.pallas_call(
        functools.partial(_tc_final_body, c_head),
        grid=(NBLK,),
        in_specs=[
            pl.BlockSpec((NC, BLK, half_in), lambda i: (0, i, 0)),
            pl.BlockSpec((BLK, LANES), lambda i: (i, 0)),
            pl.BlockSpec((1, f_in), lambda i: (0, 0)),
        ],
        out_specs=pl.BlockSpec((BLK, f_in), lambda i: (i, 0)),
        out_shape=jax.ShapeDtypeStruct((N, f_in), jnp.float32),
    )(num, den, b)


def _make_edge_kernel(f_out, heads_of_vreg_per_core):
    half = f_out // 2
    mesh = plsc.VectorSubcoreMesh(core_axis_name="c", subcore_axis_name="s",
                                  num_cores=NC, num_subcores=NS)
    body = functools.partial(_edge_body, half, heads_of_vreg_per_core)
    return pl.kernel(
        body,
        out_type=[
            jax.ShapeDtypeStruct((NC, NPAD, half), jnp.float32),
            jax.ShapeDtypeStruct((NPAD, LANES), jnp.float32),
        ],
        mesh=mesh,
        compiler_params=pltpu.CompilerParams(use_tc_tiling_on_sc=False),
        scratch_types=[
            pltpu.VMEM((CH,), jnp.int32),
            pltpu.VMEM((CH,), jnp.int32),
            pltpu.VMEM((CH,), jnp.int32),
            pltpu.VMEM((CH,), jnp.int32),
            pltpu.VMEM((CH,), jnp.int32),
            pltpu.VMEM((CH, LANES), jnp.float32),
            pltpu.VMEM((CH, LANES), jnp.float32),
            pltpu.VMEM((CH, half), jnp.float32),
            pltpu.VMEM((CH, LANES), jnp.float32),
            pltpu.VMEM((ZR, half), jnp.float32),
            pltpu.VMEM((ZR, LANES), jnp.float32),
            pltpu.SemaphoreType.DMA,
            pltpu.VMEM_SHARED((NPAD, half), jnp.float32),
            pltpu.VMEM_SHARED((NPAD, LANES), jnp.float32),
        ],
    )


def _edge_body(half, heads_of_vreg_per_core, src_hbm, dst_hbm, et_hbm,
               xw_hbm, sq_hbm, sk_hbm, num_out, den_out,
               sv, tv, dv, gsv, gdv, bufq, bufk, bufx, evb, znum, zden,
               sem, acc_num, acc_den):
    c = lax.axis_index("c")
    s = lax.axis_index("s")
    nvec = half // LANES

    zero16 = jnp.zeros((LANES,), jnp.float32)

    def _zrow(i, _):
        for j in range(nvec):
            znum[i, pl.ds(j * LANES, LANES)] = zero16
        zden[i, :] = zero16
        return 0

    lax.fori_loop(0, ZR, _zrow, 0)
    for k in range(5):
        row0 = s * RPS + k * ZR
        pltpu.sync_copy(znum, acc_num.at[pl.ds(row0, ZR)])
        pltpu.sync_copy(zden, acc_den.at[pl.ds(row0, ZR)])
    plsc.subcore_barrier()

    base_e = s * EPT
    my_xw = xw_hbm.at[c]

    def _run(head_of_vreg, scatter_den):
        def _chunk(ch, _):
            eb = base_e + ch * CH
            pltpu.sync_copy(src_hbm.at[pl.ds(eb, CH)], sv)
            pltpu.sync_copy(dst_hbm.at[pl.ds(eb, CH)], dv)
            pltpu.sync_copy(et_hbm.at[pl.ds(eb, CH)], tv)

            for j in range(CH // LANES):
                sl = pl.ds(j * LANES, LANES)
                gsv[sl] = sv[sl] * R + tv[sl]
                gdv[sl] = dv[sl] * R + tv[sl]

            cp1 = pltpu.async_copy(sq_hbm.at[gdv], bufq, sem)
            cp2 = pltpu.async_copy(sk_hbm.at[gsv], bufk, sem)
            cp3 = pltpu.async_copy(my_xw.at[gsv], bufx, sem)
            cp1.wait()
            cp2.wait()
            cp3.wait()

            def _group(g, _):
                for l in range(LANES):
                    row = g * LANES + l
                    a = bufq[row, :] + bufk[row, :]
                    a = jnp.where(a >= 0.0, a, 0.2 * a)
                    ev = jnp.exp(a)
                    if scatter_den:
                        evb[row, :] = ev
                    for j in range(nvec):
                        sl = pl.ds(j * LANES, LANES)
                        bufx[row, sl] = bufx[row, sl] * ev[head_of_vreg[j]]
                return 0

            lax.fori_loop(0, CH // LANES, _group, 0)

            if scatter_den:
                pltpu.sync_copy(evb, acc_den.at[dv], add=True)
            pltpu.sync_copy(bufx, acc_num.at[dv], add=True)
            return 0

        lax.fori_loop(0, NCHUNK, _chunk, 0)

    @pl.when(c == 0)
    def _():
        _run(heads_of_vreg_per_core[0], True)

    @pl.when(c == 1)
    def _():
        _run(heads_of_vreg_per_core[1], False)

    plsc.subcore_barrier()

    row0 = s * RPS
    pltpu.sync_copy(acc_num.at[pl.ds(row0, RPS)],
                    num_out.at[c, pl.ds(row0, RPS)])

    @pl.when(c == 0)
    def _():
        pltpu.sync_copy(acc_den.at[pl.ds(row0, RPS)],
                        den_out.at[pl.ds(row0, RPS)])


def _prep_weights(basis, comb, q, k):
    w = jnp.einsum('rb,bio->rio', comb, basis)
    f_in, f_out = w.shape[1], w.shape[2]
    half = f_out // 2
    h = q.shape[1]
    wt = w.transpose(1, 0, 2)
    wlo = wt[:, :, :half].reshape(f_in, R * half)
    whi = wt[:, :, half:].reshape(f_in, R * half)
    wq = jnp.einsum('rio,oh->rih', w, q)
    wk = jnp.einsum('rio,oh->rih', w, k)
    pad = ((0, 0), (0, 0), (0, LANES - h))
    wqp = jnp.pad(wq, pad).transpose(1, 0, 2).reshape(f_in, R * LANES)
    wkp = jnp.pad(wk, pad).transpose(1, 0, 2).reshape(f_in, R * LANES)
    return wlo, whi, wqp, wkp


def kernel(x, edge_index, edge_type, basis1, comb1, q1, k1, b1,
           basis2, comb2, q2, k2, b2, basis3, comb3, q3, k3, b3):
    src = edge_index[0]
    dst = edge_index[1]
    et = edge_type

    wlo1, whi1, wq1, wk1 = _prep_weights(basis1, comb1, q1, k1)
    wlo2, whi2, wq2, wk2 = _prep_weights(basis2, comb2, q2, k2)
    wlo3, whi3, wq3, wk3 = _prep_weights(basis3, comb3, q3, k3)

    edge1 = _make_edge_kernel(128, ((0, 0, 1, 1), (2, 2, 3, 3)))
    edge2 = _make_edge_kernel(128, ((0, 0, 0, 0), (0, 0, 0, 0)))
    edge3 = _make_edge_kernel(64, ((0, 0), (0, 0)))

    xw, sq, sk = _tc_first(x, wlo1, whi1, wq1, wk1, 128)
    num, den = edge1(src, dst, et,
                     xw.reshape(NC, N * R, 64),
                     sq.reshape(N * R, LANES), sk.reshape(N * R, LANES))
    xw, sq, sk = _tc_mid(num, den, b1.reshape(1, -1), wlo2, whi2, wq2, wk2,
                         c_head=32, f_out=128)
    num, den = edge2(src, dst, et,
                     xw.reshape(NC, N * R, 64),
                     sq.reshape(N * R, LANES), sk.reshape(N * R, LANES))
    xw, sq, sk = _tc_mid(num, den, b2.reshape(1, -1), wlo3, whi3, wq3, wk3,
                         c_head=128, f_out=64)
    num, den = edge3(src, dst, et,
                     xw.reshape(NC, N * R, 32),
                     sq.reshape(N * R, LANES), sk.reshape(N * R, LANES))
    out = _tc_final(num, den, b3.reshape(1, -1), c_head=64)
    return out


# speedup vs baseline: 14.6418x; 1.1789x over previous
import functools

import jax
import jax.numpy as jnp
from jax import lax
from jax.experimental import pallas as pl
from jax.experimental.pallas import tpu as pltpu
from jax.experimental.pallas import tpu_sc as plsc

N = 10000
E = 320000
R = 8
LANES = 16

NC = 2
NS = 16
NW = NC * NS
EPT = E // NW
CH = 80
NCHUNK = EPT // CH

BLK = 400
NBLK = N // BLK

NPAD = 10240
NPT = NPAD // NW
ACCR = NPT + 8
EPAD = E + 160

_MESH = dict(core_axis_name="c", subcore_axis_name="s",
             num_cores=NC, num_subcores=NS)


def _tc_first_body(x_ref, wf_ref, wq_ref, wk_ref, xw_ref, sq_ref, sk_ref):
    h = x_ref[...]
    xw_ref[...] = jnp.dot(h, wf_ref[...], preferred_element_type=jnp.float32)
    sq_ref[...] = jnp.dot(h, wq_ref[...], preferred_element_type=jnp.float32)
    sk_ref[...] = jnp.dot(h, wk_ref[...], preferred_element_type=jnp.float32)


def _epilogue(num_ref, den_ref, b_ref, c_head, relu):
    nb = num_ref[...]
    f_in = nb.shape[-1]
    db = den_ref[...]
    sel = (lax.broadcasted_iota(jnp.int32, (LANES, f_in), 0)
           == lax.broadcasted_iota(jnp.int32, (LANES, f_in), 1) // c_head
           ).astype(jnp.float32)
    de = jnp.dot(db, sel, preferred_element_type=jnp.float32)
    h = nb / (de + 1e-16) + b_ref[...]
    if relu:
        h = jnp.maximum(h, 0.0)
    return h


def _tc_mid_body(c_head, num_ref, den_ref, b_ref, wf_ref, wq_ref, wk_ref,
                 xw_ref, sq_ref, sk_ref):
    h = _epilogue(num_ref, den_ref, b_ref, c_head, True)
    xw_ref[...] = jnp.dot(h, wf_ref[...], preferred_element_type=jnp.float32)
    sq_ref[...] = jnp.dot(h, wq_ref[...], preferred_element_type=jnp.float32)
    sk_ref[...] = jnp.dot(h, wk_ref[...], preferred_element_type=jnp.float32)


def _tc_final_body(c_head, num_ref, den_ref, b_ref, out_ref):
    out_ref[...] = _epilogue(num_ref, den_ref, b_ref, c_head, False)


def _xw_sq_sk(f_out):
    out_specs = [
        pl.BlockSpec((BLK, R * f_out), lambda i: (i, 0)),
        pl.BlockSpec((BLK, R * LANES), lambda i: (i, 0)),
        pl.BlockSpec((BLK, R * LANES), lambda i: (i, 0)),
    ]
    out_shape = [
        jax.ShapeDtypeStruct((N, R * f_out), jnp.float32),
        jax.ShapeDtypeStruct((N, R * LANES), jnp.float32),
        jax.ShapeDtypeStruct((N, R * LANES), jnp.float32),
    ]
    return out_specs, out_shape


def _tc_first(x, wf, wq, wk, f_out):
    out_specs, out_shape = _xw_sq_sk(f_out)
    return pl.pallas_call(
        _tc_first_body,
        grid=(NBLK,),
        in_specs=[
            pl.BlockSpec((BLK, x.shape[1]), lambda i: (i, 0)),
            pl.BlockSpec(wf.shape, lambda i: (0, 0)),
            pl.BlockSpec(wq.shape, lambda i: (0, 0)),
            pl.BlockSpec(wk.shape, lambda i: (0, 0)),
        ],
        out_specs=out_specs,
        out_shape=out_shape,
    )(x, wf, wq, wk)


def _tc_mid(num, den, b, wf, wq, wk, c_head, f_out):
    f_in = num.shape[-1]
    out_specs, out_shape = _xw_sq_sk(f_out)
    return pl.pallas_call(
        functools.partial(_tc_mid_body, c_head),
        grid=(NBLK,),
        in_specs=[
            pl.BlockSpec((BLK, f_in), lambda i: (i, 0)),
            pl.BlockSpec((BLK, LANES), lambda i: (i, 0)),
            pl.BlockSpec((1, f_in), lambda i: (0, 0)),
            pl.BlockSpec(wf.shape, lambda i: (0, 0)),
            pl.BlockSpec(wq.shape, lambda i: (0, 0)),
            pl.BlockSpec(wk.shape, lambda i: (0, 0)),
        ],
        out_specs=out_specs,
        out_shape=out_shape,
    )(num, den, b, wf, wq, wk)


def _tc_final(num, den, b, c_head):
    f_in = num.shape[-1]
    return pl.pallas_call(
        functools.partial(_tc_final_body, c_head),
        grid=(NBLK,),
        in_specs=[
            pl.BlockSpec((BLK, f_in), lambda i: (i, 0)),
            pl.BlockSpec((BLK, LANES), lambda i: (i, 0)),
            pl.BlockSpec((1, f_in), lambda i: (0, 0)),
        ],
        out_specs=pl.BlockSpec((BLK, f_in), lambda i: (i, 0)),
        out_shape=jax.ShapeDtypeStruct((N, f_in), jnp.float32),
    )(num, den, b)


def _wid():
    return lax.axis_index("c") * NS + lax.axis_index("s")


def _bucket_of(dv):
    return (dv * 6554) >> 21


def _count_body(ei_hbm, et_hbm, cnt_out, dv, cbuf, csm, sem):
    g = _wid()
    iota = lax.iota(jnp.int32, LANES)
    base_e = g * EPT

    for b in range(NW):
        csm[b] = 0

    def _chunk(ch, _):
        pltpu.sync_copy(ei_hbm.at[1, pl.ds(base_e + ch * CH, CH)], dv)
        for j in range(CH // LANES):
            bv = _bucket_of(dv[pl.ds(j * LANES, LANES)])
            for l in range(LANES):
                b = bv[l]
                csm[b] = csm[b] + 1
        return 0

    lax.fori_loop(0, NCHUNK, _chunk, 0)
    c0 = jnp.zeros((LANES,), jnp.int32)
    c1 = jnp.zeros((LANES,), jnp.int32)
    for b in range(NW):
        onehot = jnp.where(iota == (b % LANES), csm[b], 0)
        if b < LANES:
            c0 = c0 + onehot
        else:
            c1 = c1 + onehot
    cbuf[pl.ds(0, LANES)] = c0
    cbuf[pl.ds(LANES, LANES)] = c1
    pltpu.sync_copy(cbuf, cnt_out.at[g])


def _make_count_kernel():
    return pl.kernel(
        _count_body,
        out_type=jax.ShapeDtypeStruct((NW, NW), jnp.int32),
        mesh=plsc.VectorSubcoreMesh(**_MESH),
        compiler_params=pltpu.CompilerParams(use_tc_tiling_on_sc=False),
        scratch_types=[
            pltpu.VMEM((CH,), jnp.int32),
            pltpu.VMEM((NW,), jnp.int32),
            pltpu.SMEM((NW,), jnp.int32),
            pltpu.SemaphoreType.DMA,
        ],
    )


def _counts_to_scalars(cnt_hbm, cbuf):
    pltpu.sync_copy(cnt_hbm, cbuf)
    zero = jnp.zeros((LANES,), jnp.int32)

    def _acc(t, carry):
        a0, a1 = carry
        return a0 + cbuf[t, pl.ds(0, LANES)], a1 + cbuf[t, pl.ds(LANES, LANES)]

    t0, t1 = lax.fori_loop(0, NW, _acc, (zero, zero))
    tot = [(t0 if b < LANES else t1)[b % LANES] for b in range(NW)]
    starts = []
    s = jnp.int32(0)
    for b in range(NW):
        starts.append(s)
        s = s + tot[b]
    return tot, starts


def _select_scalar(vals, idx):
    out = jnp.int32(0)
    for b, v in enumerate(vals):
        out = out + jnp.where(idx == b, v, 0)
    return out


def _scatter_body(ei_hbm, et_hbm, cnt_hbm, ps_out, pd_out,
                  sv, dv, tv, psb, pdb, posb, cbuf, bsm, sem):
    g = _wid()
    iota = lax.iota(jnp.int32, LANES)
    _, starts = _counts_to_scalars(cnt_hbm, cbuf)

    def _lower(t, carry):
        a0, a1 = carry
        take = t < g
        r0 = jnp.where(take, cbuf[t, pl.ds(0, LANES)], 0)
        r1 = jnp.where(take, cbuf[t, pl.ds(LANES, LANES)], 0)
        return a0 + r0, a1 + r1

    zero = jnp.zeros((LANES,), jnp.int32)
    p0, p1 = lax.fori_loop(0, NW, _lower, (zero, zero))
    for b in range(NW):
        bsm[b] = starts[b] + (p0 if b < LANES else p1)[b % LANES]

    base_e = g * EPT

    def _chunk(ch, _):
        eb = base_e + ch * CH
        pltpu.sync_copy(ei_hbm.at[0, pl.ds(eb, CH)], sv)
        pltpu.sync_copy(ei_hbm.at[1, pl.ds(eb, CH)], dv)
        pltpu.sync_copy(et_hbm.at[pl.ds(eb, CH)], tv)
        for j in range(CH // LANES):
            sl = pl.ds(j * LANES, LANES)
            dv16 = dv[sl]
            bv = _bucket_of(dv16)
            pos = jnp.zeros((LANES,), jnp.int32)
            for l in range(LANES):
                b = bv[l]
                p = bsm[b]
                bsm[b] = p + 1
                pos = pos + jnp.where(iota == l, p, 0)
            psb[sl] = sv[sl] * R + tv[sl]
            pdb[sl] = dv16
            posb[sl] = pos
        cp1 = pltpu.async_copy(psb, ps_out.at[posb], sem)
        cp2 = pltpu.async_copy(pdb, pd_out.at[posb], sem)
        cp1.wait()
        cp2.wait()
        return 0

    lax.fori_loop(0, NCHUNK, _chunk, 0)


def _make_scatter_kernel():
    return pl.kernel(
        _scatter_body,
        out_type=[
            jax.ShapeDtypeStruct((EPAD,), jnp.int32),
            jax.ShapeDtypeStruct((EPAD,), jnp.int32),
        ],
        mesh=plsc.VectorSubcoreMesh(**_MESH),
        compiler_params=pltpu.CompilerParams(use_tc_tiling_on_sc=False),
        scratch_types=[
            pltpu.VMEM((CH,), jnp.int32),
            pltpu.VMEM((CH,), jnp.int32),
            pltpu.VMEM((CH,), jnp.int32),
            pltpu.VMEM((CH,), jnp.int32),
            pltpu.VMEM((CH,), jnp.int32),
            pltpu.VMEM((CH,), jnp.int32),
            pltpu.VMEM((NW, NW), jnp.int32),
            pltpu.SMEM((NW,), jnp.int32),
            pltpu.SemaphoreType.DMA,
        ],
    )


def _edge_body(f_out, head_of_vreg, ps_hbm, pd_hbm, cnt_hbm,
               xw_hbm, sq_hbm, sk_hbm, num_out, den_out,
               psv, pdv, gsb, gdb, lrb, bufq, bufk, bufx, cbuf,
               sem, acc, accd):
    g = _wid()
    iota = lax.iota(jnp.int32, LANES)
    nvec = f_out // LANES

    zero16 = jnp.zeros((LANES,), jnp.float32)

    def _zrow(i, _):
        for j in range(nvec):
            acc[i, pl.ds(j * LANES, LANES)] = zero16
        accd[i, :] = zero16
        return 0

    lax.fori_loop(0, ACCR, _zrow, 0)

    tot, starts = _counts_to_scalars(cnt_hbm, cbuf)
    start = _select_scalar(starts, g)
    total = _select_scalar(tot, g)
    end = start + total
    base_al = pl.multiple_of((start >> 3) << 3, 8)
    nch = (((end - base_al + CH - 1) >> 4) * 26215) >> 17
    gbase = g * NPT

    def _chunk(ch, _):
        eb = base_al + ch * CH
        pltpu.sync_copy(ps_hbm.at[pl.ds(eb, CH)], psv)
        pltpu.sync_copy(pd_hbm.at[pl.ds(eb, CH)], pdv)

        for j in range(CH // LANES):
            sl = pl.ds(j * LANES, LANES)
            pos = iota + (eb + j * LANES)
            valid = (pos >= start) & (pos < end)
            ps16 = jnp.where(valid, psv[sl], 0)
            pd16 = jnp.where(valid, pdv[sl], gbase)
            gsb[sl] = ps16
            gdb[sl] = pd16 * R + (ps16 & (R - 1))
            lrb[sl] = jnp.where(valid, pd16 - gbase, NPT)

        cp1 = pltpu.async_copy(sq_hbm.at[gdb], bufq, sem)
        cp2 = pltpu.async_copy(sk_hbm.at[gsb], bufk, sem)
        cp3 = pltpu.async_copy(xw_hbm.at[gsb], bufx, sem)
        cp1.wait()
        cp2.wait()
        cp3.wait()

        def _group(j, _):
            lrv = lrb[pl.ds(j * LANES, LANES)]
            for l in range(LANES):
                row = j * LANES + l
                lr = lrv[l]
                a = bufq[row, :] + bufk[row, :]
                a = jnp.where(a >= 0.0, a, 0.2 * a)
                ev = jnp.exp(a)
                plsc.addupdate(accd.at[lr, :], ev)
                for jj in range(nvec):
                    sl = pl.ds(jj * LANES, LANES)
                    plsc.addupdate(acc.at[lr, sl],
                                   bufx[row, sl] * ev[head_of_vreg[jj]])
            return 0

        lax.fori_loop(0, CH // LANES, _group, 0)
        return 0

    lax.fori_loop(0, nch, _chunk, 0)

    pltpu.sync_copy(acc.at[pl.ds(0, NPT)], num_out.at[pl.ds(gbase, NPT)])
    pltpu.sync_copy(accd.at[pl.ds(0, NPT)], den_out.at[pl.ds(gbase, NPT)])


def _make_edge_kernel(f_out, head_of_vreg):
    body = functools.partial(_edge_body, f_out, head_of_vreg)
    return pl.kernel(
        body,
        out_type=[
            jax.ShapeDtypeStruct((NPAD, f_out), jnp.float32),
            jax.ShapeDtypeStruct((NPAD, LANES), jnp.float32),
        ],
        mesh=plsc.VectorSubcoreMesh(**_MESH),
        compiler_params=pltpu.CompilerParams(use_tc_tiling_on_sc=False),
        scratch_types=[
            pltpu.VMEM((CH,), jnp.int32),
            pltpu.VMEM((CH,), jnp.int32),
            pltpu.VMEM((CH,), jnp.int32),
            pltpu.VMEM((CH,), jnp.int32),
            pltpu.VMEM((CH,), jnp.int32),
            pltpu.VMEM((CH, LANES), jnp.float32),
            pltpu.VMEM((CH, LANES), jnp.float32),
            pltpu.VMEM((CH, f_out), jnp.float32),
            pltpu.VMEM((NW, NW), jnp.int32),
            pltpu.SemaphoreType.DMA,
            pltpu.VMEM((ACCR, f_out), jnp.float32),
            pltpu.VMEM((ACCR, LANES), jnp.float32),
        ],
    )


def _prep_weights(basis, comb, q, k):
    w = jnp.einsum('rb,bio->rio', comb, basis)
    f_in = w.shape[1]
    h = q.shape[1]
    wf = w.transpose(1, 0, 2).reshape(f_in, R * w.shape[2])
    wq = jnp.einsum('rio,oh->rih', w, q)
    wk = jnp.einsum('rio,oh->rih', w, k)
    pad = ((0, 0), (0, 0), (0, LANES - h))
    wqp = jnp.pad(wq, pad).transpose(1, 0, 2).reshape(f_in, R * LANES)
    wkp = jnp.pad(wk, pad).transpose(1, 0, 2).reshape(f_in, R * LANES)
    return wf, wqp, wkp


def kernel(x, edge_index, edge_type, basis1, comb1, q1, k1, b1,
           basis2, comb2, q2, k2, b2, basis3, comb3, q3, k3, b3):
    wf1, wq1, wk1 = _prep_weights(basis1, comb1, q1, k1)
    wf2, wq2, wk2 = _prep_weights(basis2, comb2, q2, k2)
    wf3, wq3, wk3 = _prep_weights(basis3, comb3, q3, k3)

    counts = _make_count_kernel()(edge_index, edge_type)
    ps, pd = _make_scatter_kernel()(edge_index, edge_type, counts)

    edge1 = _make_edge_kernel(128, (0, 0, 1, 1, 2, 2, 3, 3))
    edge2 = _make_edge_kernel(128, (0,) * 8)
    edge3 = _make_edge_kernel(64, (0,) * 4)

    xw, sq, sk = _tc_first(x, wf1, wq1, wk1, 128)
    num, den = edge1(ps, pd, counts, xw.reshape(N * R, 128),
                     sq.reshape(N * R, LANES), sk.reshape(N * R, LANES))
    xw, sq, sk = _tc_mid(num, den, b1.reshape(1, -1), wf2, wq2, wk2,
                         c_head=32, f_out=128)
    num, den = edge2(ps, pd, counts, xw.reshape(N * R, 128),
                     sq.reshape(N * R, LANES), sk.reshape(N * R, LANES))
    xw, sq, sk = _tc_mid(num, den, b2.reshape(1, -1), wf3, wq3, wk3,
                         c_head=128, f_out=64)
    num, den = edge3(ps, pd, counts, xw.reshape(N * R, 64),
                     sq.reshape(N * R, LANES), sk.reshape(N * R, LANES))
    out = _tc_final(num, den, b3.reshape(1, -1), c_head=64)
    return out


# speedup vs baseline: 15.7821x; 1.0779x over previous
import functools

import jax
import jax.numpy as jnp
from jax import lax
from jax.experimental import pallas as pl
from jax.experimental.pallas import tpu as pltpu
from jax.experimental.pallas import tpu_sc as plsc

N = 10000
E = 320000
R = 8
LANES = 16

NC = 2
NS = 16
NW = NC * NS
EPT = E // NW
CH = 80
NCHUNK = EPT // CH
ECH = 128

BLK = 400
NBLK = N // BLK

NPAD = 10240
NPT = NPAD // NW
ACCR = NPT + 8
EPAD = E + 384

_MESH = dict(core_axis_name="c", subcore_axis_name="s",
             num_cores=NC, num_subcores=NS)


def _tc_first_body(x_ref, wf_ref, wq_ref, wk_ref, xw_ref, sq_ref, sk_ref):
    h = x_ref[...]
    xw_ref[...] = jnp.dot(h, wf_ref[...], preferred_element_type=jnp.float32)
    sq_ref[...] = jnp.dot(h, wq_ref[...], preferred_element_type=jnp.float32)
    sk_ref[...] = jnp.dot(h, wk_ref[...], preferred_element_type=jnp.float32)


def _epilogue(num_ref, den_ref, b_ref, c_head, relu):
    nb = num_ref[...]
    f_in = nb.shape[-1]
    db = den_ref[...]
    sel = (lax.broadcasted_iota(jnp.int32, (LANES, f_in), 0)
           == lax.broadcasted_iota(jnp.int32, (LANES, f_in), 1) // c_head
           ).astype(jnp.float32)
    de = jnp.dot(db, sel, preferred_element_type=jnp.float32)
    h = nb / (de + 1e-16) + b_ref[...]
    if relu:
        h = jnp.maximum(h, 0.0)
    return h


def _tc_mid_body(c_head, num_ref, den_ref, b_ref, wf_ref, wq_ref, wk_ref,
                 xw_ref, sq_ref, sk_ref):
    h = _epilogue(num_ref, den_ref, b_ref, c_head, True)
    xw_ref[...] = jnp.dot(h, wf_ref[...], preferred_element_type=jnp.float32)
    sq_ref[...] = jnp.dot(h, wq_ref[...], preferred_element_type=jnp.float32)
    sk_ref[...] = jnp.dot(h, wk_ref[...], preferred_element_type=jnp.float32)


def _tc_final_body(c_head, num_ref, den_ref, b_ref, out_ref):
    out_ref[...] = _epilogue(num_ref, den_ref, b_ref, c_head, False)


def _xw_sq_sk(f_out):
    out_specs = [
        pl.BlockSpec((BLK, R * f_out), lambda i: (i, 0)),
        pl.BlockSpec((BLK, R * LANES), lambda i: (i, 0)),
        pl.BlockSpec((BLK, R * LANES), lambda i: (i, 0)),
    ]
    out_shape = [
        jax.ShapeDtypeStruct((N, R * f_out), jnp.float32),
        jax.ShapeDtypeStruct((N, R * LANES), jnp.float32),
        jax.ShapeDtypeStruct((N, R * LANES), jnp.float32),
    ]
    return out_specs, out_shape


def _tc_first(x, wf, wq, wk, f_out):
    out_specs, out_shape = _xw_sq_sk(f_out)
    return pl.pallas_call(
        _tc_first_body,
        grid=(NBLK,),
        in_specs=[
            pl.BlockSpec((BLK, x.shape[1]), lambda i: (i, 0)),
            pl.BlockSpec(wf.shape, lambda i: (0, 0)),
            pl.BlockSpec(wq.shape, lambda i: (0, 0)),
            pl.BlockSpec(wk.shape, lambda i: (0, 0)),
        ],
        out_specs=out_specs,
        out_shape=out_shape,
    )(x, wf, wq, wk)


def _tc_mid(num, den, b, wf, wq, wk, c_head, f_out):
    f_in = num.shape[-1]
    out_specs, out_shape = _xw_sq_sk(f_out)
    return pl.pallas_call(
        functools.partial(_tc_mid_body, c_head),
        grid=(NBLK,),
        in_specs=[
            pl.BlockSpec((BLK, f_in), lambda i: (i, 0)),
            pl.BlockSpec((BLK, LANES), lambda i: (i, 0)),
            pl.BlockSpec((1, f_in), lambda i: (0, 0)),
            pl.BlockSpec(wf.shape, lambda i: (0, 0)),
            pl.BlockSpec(wq.shape, lambda i: (0, 0)),
            pl.BlockSpec(wk.shape, lambda i: (0, 0)),
        ],
        out_specs=out_specs,
        out_shape=out_shape,
    )(num, den, b, wf, wq, wk)


def _tc_final(num, den, b, c_head):
    f_in = num.shape[-1]
    return pl.pallas_call(
        functools.partial(_tc_final_body, c_head),
        grid=(NBLK,),
        in_specs=[
            pl.BlockSpec((BLK, f_in), lambda i: (i, 0)),
            pl.BlockSpec((BLK, LANES), lambda i: (i, 0)),
            pl.BlockSpec((1, f_in), lambda i: (0, 0)),
        ],
        out_specs=pl.BlockSpec((BLK, f_in), lambda i: (i, 0)),
        out_shape=jax.ShapeDtypeStruct((N, f_in), jnp.float32),
    )(num, den, b)


def _wid():
    return lax.axis_index("c") * NS + lax.axis_index("s")


def _bucket_of(dv):
    return (dv * 6554) >> 21


def _count_body(ei_hbm, et_hbm, cnt_out, dv, cbuf, csm, sem):
    g = _wid()
    iota = lax.iota(jnp.int32, LANES)
    base_e = g * EPT

    for b in range(NW):
        csm[b] = 0

    def _chunk(ch, _):
        pltpu.sync_copy(ei_hbm.at[1, pl.ds(base_e + ch * CH, CH)], dv)
        for j in range(CH // LANES):
            bv = _bucket_of(dv[pl.ds(j * LANES, LANES)])
            for l in range(LANES):
                b = bv[l]
                csm[b] = csm[b] + 1
        return 0

    lax.fori_loop(0, NCHUNK, _chunk, 0)
    c0 = jnp.zeros((LANES,), jnp.int32)
    c1 = jnp.zeros((LANES,), jnp.int32)
    for b in range(NW):
        onehot = jnp.where(iota == (b % LANES), csm[b], 0)
        if b < LANES:
            c0 = c0 + onehot
        else:
            c1 = c1 + onehot
    cbuf[pl.ds(0, LANES)] = c0
    cbuf[pl.ds(LANES, LANES)] = c1
    pltpu.sync_copy(cbuf, cnt_out.at[g])


def _make_count_kernel():
    return pl.kernel(
        _count_body,
        out_type=jax.ShapeDtypeStruct((NW, NW), jnp.int32),
        mesh=plsc.VectorSubcoreMesh(**_MESH),
        compiler_params=pltpu.CompilerParams(use_tc_tiling_on_sc=False),
        scratch_types=[
            pltpu.VMEM((CH,), jnp.int32),
            pltpu.VMEM((NW,), jnp.int32),
            pltpu.SMEM((NW,), jnp.int32),
            pltpu.SemaphoreType.DMA,
        ],
    )


def _counts_to_scalars(cnt_hbm, cbuf):
    pltpu.sync_copy(cnt_hbm, cbuf)
    zero = jnp.zeros((LANES,), jnp.int32)

    def _acc(t, carry):
        a0, a1 = carry
        return a0 + cbuf[t, pl.ds(0, LANES)], a1 + cbuf[t, pl.ds(LANES, LANES)]

    t0, t1 = lax.fori_loop(0, NW, _acc, (zero, zero))
    tot = [(t0 if b < LANES else t1)[b % LANES] for b in range(NW)]
    starts = []
    s = jnp.int32(0)
    for b in range(NW):
        starts.append(s)
        s = s + tot[b]
    return tot, starts


def _select_scalar(vals, idx):
    out = jnp.int32(0)
    for b, v in enumerate(vals):
        out = out + jnp.where(idx == b, v, 0)
    return out


def _scatter_body(ei_hbm, et_hbm, cnt_hbm, ps_out, pd_out,
                  sv, dv, tv, psb, pdb, posb, cbuf, bsm, sem):
    g = _wid()
    iota = lax.iota(jnp.int32, LANES)
    _, starts = _counts_to_scalars(cnt_hbm, cbuf)

    def _lower(t, carry):
        a0, a1 = carry
        take = t < g
        r0 = jnp.where(take, cbuf[t, pl.ds(0, LANES)], 0)
        r1 = jnp.where(take, cbuf[t, pl.ds(LANES, LANES)], 0)
        return a0 + r0, a1 + r1

    zero = jnp.zeros((LANES,), jnp.int32)
    p0, p1 = lax.fori_loop(0, NW, _lower, (zero, zero))
    for b in range(NW):
        bsm[b] = starts[b] + (p0 if b < LANES else p1)[b % LANES]

    base_e = g * EPT

    def _chunk(ch, _):
        eb = base_e + ch * CH
        pltpu.sync_copy(ei_hbm.at[0, pl.ds(eb, CH)], sv)
        pltpu.sync_copy(ei_hbm.at[1, pl.ds(eb, CH)], dv)
        pltpu.sync_copy(et_hbm.at[pl.ds(eb, CH)], tv)
        for j in range(CH // LANES):
            sl = pl.ds(j * LANES, LANES)
            dv16 = dv[sl]
            bv = _bucket_of(dv16)
            pos = jnp.zeros((LANES,), jnp.int32)
            for l in range(LANES):
                b = bv[l]
                p = bsm[b]
                bsm[b] = p + 1
                pos = pos + jnp.where(iota == l, p, 0)
            psb[sl] = sv[sl] * R + tv[sl]
            pdb[sl] = dv16
            posb[sl] = pos
        cp1 = pltpu.async_copy(psb, ps_out.at[posb], sem)
        cp2 = pltpu.async_copy(pdb, pd_out.at[posb], sem)
        cp1.wait()
        cp2.wait()
        return 0

    lax.fori_loop(0, NCHUNK, _chunk, 0)


def _make_scatter_kernel():
    return pl.kernel(
        _scatter_body,
        out_type=[
            jax.ShapeDtypeStruct((EPAD,), jnp.int32),
            jax.ShapeDtypeStruct((EPAD,), jnp.int32),
        ],
        mesh=plsc.VectorSubcoreMesh(**_MESH),
        compiler_params=pltpu.CompilerParams(use_tc_tiling_on_sc=False),
        scratch_types=[
            pltpu.VMEM((CH,), jnp.int32),
            pltpu.VMEM((CH,), jnp.int32),
            pltpu.VMEM((CH,), jnp.int32),
            pltpu.VMEM((CH,), jnp.int32),
            pltpu.VMEM((CH,), jnp.int32),
            pltpu.VMEM((CH,), jnp.int32),
            pltpu.VMEM((NW, NW), jnp.int32),
            pltpu.SMEM((NW,), jnp.int32),
            pltpu.SemaphoreType.DMA,
        ],
    )


def _edge_chunk(eb, start, end, gbase, ps_hbm, pd_hbm, psv, pdv,
                gsb, gdb, lrb, semi):
    c1 = pltpu.async_copy(ps_hbm.at[pl.ds(eb, ECH)], psv, semi)
    c2 = pltpu.async_copy(pd_hbm.at[pl.ds(eb, ECH)], pdv, semi)
    return c1, c2


def _edge_body(f_out, head_of_vreg, ps_hbm, pd_hbm, cnt_hbm,
               xw_hbm, sq_hbm, sk_hbm, num_out, den_out,
               psvA, pdvA, gsbA, gdbA, lrbA, bufqA, bufkA, bufxA,
               psvB, pdvB, gsbB, gdbB, lrbB, bufqB, bufkB, bufxB,
               cbuf, semIA, semIB, semGA, semGB, acc, accd):
    g = _wid()
    iota = lax.iota(jnp.int32, LANES)
    nvec = f_out // LANES

    zero16 = jnp.zeros((LANES,), jnp.float32)

    def _zrow(i, _):
        for j in range(nvec):
            acc[i, pl.ds(j * LANES, LANES)] = zero16
        accd[i, :] = zero16
        return 0

    lax.fori_loop(0, ACCR, _zrow, 0)

    tot, starts = _counts_to_scalars(cnt_hbm, cbuf)
    start = _select_scalar(starts, g)
    total = _select_scalar(tot, g)
    end = start + total
    base_al = pl.multiple_of((start >> 3) << 3, 8)
    nch = (end - base_al + ECH - 1) >> 7
    npair = (nch + 1) >> 1
    gbase = g * NPT

    def _mask_idx(eb, psv, pdv, gsb, gdb, lrb):
        for j in range(ECH // LANES):
            sl = pl.ds(j * LANES, LANES)
            pos = iota + (eb + j * LANES)
            valid = (pos >= start) & (pos < end)
            ps16 = jnp.where(valid, psv[sl], 0)
            pd16 = jnp.where(valid, pdv[sl], gbase)
            gsb[sl] = ps16
            gdb[sl] = pd16 * R + (ps16 & (R - 1))
            lrb[sl] = jnp.where(valid, pd16 - gbase, NPT)

    def _fire_gathers(gsb, gdb, bufq, bufk, bufx, semg):
        c1 = pltpu.async_copy(sq_hbm.at[gdb], bufq, semg)
        c2 = pltpu.async_copy(sk_hbm.at[gsb], bufk, semg)
        c3 = pltpu.async_copy(xw_hbm.at[gsb], bufx, semg)
        return c1, c2, c3

    def _accumulate(lrb, bufq, bufk, bufx):
        def _group(j, _):
            lrv = lrb[pl.ds(j * LANES, LANES)]
            for l in range(LANES):
                row = j * LANES + l
                lr = lrv[l]
                a = bufq[row, :] + bufk[row, :]
                a = jnp.where(a >= 0.0, a, 0.2 * a)
                ev = jnp.exp(a)
                plsc.addupdate(accd.at[lr, :], ev)
                for jj in range(nvec):
                    sl = pl.ds(jj * LANES, LANES)
                    plsc.addupdate(acc.at[lr, sl],
                                   bufx[row, sl] * ev[head_of_vreg[jj]])
            return 0

        lax.fori_loop(0, ECH // LANES, _group, 0)

    def _pair(it, _):
        ebA = base_al + it * (2 * ECH)
        ebB = ebA + ECH
        iA1 = pltpu.async_copy(ps_hbm.at[pl.ds(ebA, ECH)], psvA, semIA)
        iA2 = pltpu.async_copy(pd_hbm.at[pl.ds(ebA, ECH)], pdvA, semIA)
        iB1 = pltpu.async_copy(ps_hbm.at[pl.ds(ebB, ECH)], psvB, semIB)
        iB2 = pltpu.async_copy(pd_hbm.at[pl.ds(ebB, ECH)], pdvB, semIB)
        iA1.wait()
        iA2.wait()
        _mask_idx(ebA, psvA, pdvA, gsbA, gdbA, lrbA)
        gA = _fire_gathers(gsbA, gdbA, bufqA, bufkA, bufxA, semGA)
        iB1.wait()
        iB2.wait()
        _mask_idx(ebB, psvB, pdvB, gsbB, gdbB, lrbB)
        gB = _fire_gathers(gsbB, gdbB, bufqB, bufkB, bufxB, semGB)
        for c in gA:
            c.wait()
        _accumulate(lrbA, bufqA, bufkA, bufxA)
        for c in gB:
            c.wait()
        _accumulate(lrbB, bufqB, bufkB, bufxB)
        return 0

    lax.fori_loop(0, npair, _pair, 0)

    pltpu.sync_copy(acc.at[pl.ds(0, NPT)], num_out.at[pl.ds(gbase, NPT)])
    pltpu.sync_copy(accd.at[pl.ds(0, NPT)], den_out.at[pl.ds(gbase, NPT)])


def _make_edge_kernel(f_out, head_of_vreg):
    body = functools.partial(_edge_body, f_out, head_of_vreg)
    idx_bufs = [pltpu.VMEM((ECH,), jnp.int32)] * 5
    dat_bufs = [
        pltpu.VMEM((ECH, LANES), jnp.float32),
        pltpu.VMEM((ECH, LANES), jnp.float32),
        pltpu.VMEM((ECH, f_out), jnp.float32),
    ]
    return pl.kernel(
        body,
        out_type=[
            jax.ShapeDtypeStruct((NPAD, f_out), jnp.float32),
            jax.ShapeDtypeStruct((NPAD, LANES), jnp.float32),
        ],
        mesh=plsc.VectorSubcoreMesh(**_MESH),
        compiler_params=pltpu.CompilerParams(use_tc_tiling_on_sc=False),
        scratch_types=(
            idx_bufs + dat_bufs + idx_bufs + dat_bufs
            + [
                pltpu.VMEM((NW, NW), jnp.int32),
                pltpu.SemaphoreType.DMA,
                pltpu.SemaphoreType.DMA,
                pltpu.SemaphoreType.DMA,
                pltpu.SemaphoreType.DMA,
                pltpu.VMEM((ACCR, f_out), jnp.float32),
                pltpu.VMEM((ACCR, LANES), jnp.float32),
            ]
        ),
    )


def _prep_weights(basis, comb, q, k):
    w = jnp.einsum('rb,bio->rio', comb, basis)
    f_in = w.shape[1]
    h = q.shape[1]
    wf = w.transpose(1, 0, 2).reshape(f_in, R * w.shape[2])
    wq = jnp.einsum('rio,oh->rih', w, q)
    wk = jnp.einsum('rio,oh->rih', w, k)
    pad = ((0, 0), (0, 0), (0, LANES - h))
    wqp = jnp.pad(wq, pad).transpose(1, 0, 2).reshape(f_in, R * LANES)
    wkp = jnp.pad(wk, pad).transpose(1, 0, 2).reshape(f_in, R * LANES)
    return wf, wqp, wkp


def kernel(x, edge_index, edge_type, basis1, comb1, q1, k1, b1,
           basis2, comb2, q2, k2, b2, basis3, comb3, q3, k3, b3):
    wf1, wq1, wk1 = _prep_weights(basis1, comb1, q1, k1)
    wf2, wq2, wk2 = _prep_weights(basis2, comb2, q2, k2)
    wf3, wq3, wk3 = _prep_weights(basis3, comb3, q3, k3)

    counts = _make_count_kernel()(edge_index, edge_type)
    ps, pd = _make_scatter_kernel()(edge_index, edge_type, counts)

    edge1 = _make_edge_kernel(128, (0, 0, 1, 1, 2, 2, 3, 3))
    edge2 = _make_edge_kernel(128, (0,) * 8)
    edge3 = _make_edge_kernel(64, (0,) * 4)

    xw, sq, sk = _tc_first(x, wf1, wq1, wk1, 128)
    num, den = edge1(ps, pd, counts, xw.reshape(N * R, 128),
                     sq.reshape(N * R, LANES), sk.reshape(N * R, LANES))
    xw, sq, sk = _tc_mid(num, den, b1.reshape(1, -1), wf2, wq2, wk2,
                         c_head=32, f_out=128)
    num, den = edge2(ps, pd, counts, xw.reshape(N * R, 128),
                     sq.reshape(N * R, LANES), sk.reshape(N * R, LANES))
    xw, sq, sk = _tc_mid(num, den, b2.reshape(1, -1), wf3, wq3, wk3,
                         c_head=128, f_out=64)
    num, den = edge3(ps, pd, counts, xw.reshape(N * R, 64),
                     sq.reshape(N * R, LANES), sk.reshape(N * R, LANES))
    out = _tc_final(num, den, b3.reshape(1, -1), c_head=64)
    return out
